# baseline (speedup 1.0000x reference)
import functools

import jax
import jax.numpy as jnp
from jax import lax
from jax.experimental import pallas as pl
from jax.experimental.pallas import tpu as pltpu
from jax.experimental.pallas import tpu_sc as plsc

DIM = 64
EPS = 1e-12
B, S = 4096, 200
NC, NS = 2, 16
NW = NC * NS
CHUNK_SEQ = 2
CT = CHUNK_SEQ * S
NCHUNK = B // CHUNK_SEQ
CPW = NCHUNK // NW
NP = CPW // 2
NIDX = 4
IDXW = CT // NIDX
LANES = 16
NV = DIM // LANES
GPC = CT // LANES


def _rsqrt(v):
    vi = lax.bitcast_convert_type(v, jnp.int32)
    yi = jnp.int32(0x5F3759DF) - lax.shift_right_arithmetic(vi, 1)
    y = lax.bitcast_convert_type(yi, jnp.float32)
    h = v * 0.5
    for _ in range(3):
        y = y * (1.5 - h * y * y)
    return y


def kernel(x, W, P, gamma, beta):
    x = x.astype(jnp.int32).reshape(NCHUNK, NIDX, IDXW)
    mesh = plsc.VectorSubcoreMesh(core_axis_name="c", subcore_axis_name="s")

    @functools.partial(
        pl.kernel,
        out_type=jax.ShapeDtypeStruct((NCHUNK, S, 2 * DIM), jnp.float32),
        mesh=mesh,
        scratch_types=[
            pltpu.VMEM((NIDX, IDXW), jnp.int32),
            pltpu.VMEM((NIDX, IDXW), jnp.int32),
            pltpu.VMEM((CT, DIM), jnp.float32),
            pltpu.VMEM((S, 2 * DIM), jnp.float32),
            pltpu.VMEM((S, 2 * DIM), jnp.float32),
            pltpu.VMEM((S, DIM), jnp.float32),
            pltpu.VMEM((DIM,), jnp.float32),
            pltpu.VMEM((DIM,), jnp.float32),
            pltpu.VMEM((CT * 17,), jnp.float32),
            pltpu.VMEM((CT * 17,), jnp.float32),
            pltpu.SemaphoreType.DMA,
            pltpu.SemaphoreType.DMA,
            pltpu.SemaphoreType.DMA,
            pltpu.SemaphoreType.DMA,
            pltpu.SemaphoreType.DMA,
        ],
        compiler_params=pltpu.CompilerParams(needs_layout_passes=False,
                                             use_tc_tiling_on_sc=False),
    )
    def sc_fn(x_hbm, w_hbm, p_hbm, g_hbm, b_hbm, out_hbm,
              idx0, idx1, gbuf, wbuf0, wbuf1, p_v, g_v, b_v,
              sbuf, qbuf, gsem, osem0, osem1, isem0, isem1):
        wid = lax.axis_index("s") * NC + lax.axis_index("c")
        base = wid * CPW
        pltpu.sync_copy(p_hbm, p_v)
        pltpu.sync_copy(g_hbm, g_v)
        pltpu.sync_copy(b_hbm, b_v)
        g_regs = [g_v[pl.ds(c * LANES, LANES)] for c in range(NV)]
        b_regs = [b_v[pl.ds(c * LANES, LANES)] for c in range(NV)]

        def fire_gather(idx_b):
            for j in range(NIDX):
                pltpu.async_copy(w_hbm.at[idx_b.at[j]],
                                 gbuf.at[pl.ds(j * IDXW, IDXW)], gsem)

        def drain_gather(idx_b):
            for j in range(NIDX):
                pltpu.make_async_copy(w_hbm.at[idx_b.at[j]],
                                      gbuf.at[pl.ds(j * IDXW, IDXW)],
                                      gsem).wait()

        def fire_idx(cg, idx_b, isem):
            pltpu.async_copy(x_hbm.at[cg], idx_b, isem)

        def drain_idx(cg, idx_b, isem):
            pltpu.make_async_copy(x_hbm.at[cg], idx_b, isem).wait()

        iota17 = lax.iota(jnp.int32, LANES) * 17

        def pass1(wbuf):
            @plsc.parallel_loop(0, S, unroll=4)
            def body(si):
                p = [p_v[si, pl.ds(c * LANES, LANES)] for c in range(NV)]
                for r2 in range(CHUNK_SEQ):
                    t = r2 * S + si
                    e = [gbuf[t, pl.ds(c * LANES, LANES)] + p[c]
                         for c in range(NV)]
                    for c in range(NV):
                        wbuf[si, pl.ds(r2 * DIM + c * LANES, LANES)] = e[c]
                    s4 = (e[0] + e[1]) + (e[2] + e[3])
                    q4 = (e[0] * e[0] + e[1] * e[1]) + (e[2] * e[2] + e[3] * e[3])
                    sbuf[pl.ds(t * 17, LANES)] = s4
                    qbuf[pl.ds(t * 17, LANES)] = q4

        def pass2(wbuf):
            @plsc.parallel_loop(0, GPC, unroll=2)
            def body(k):
                bvec = iota17 + k * (LANES * 17)
                stot = plsc.load_gather(sbuf, [bvec])
                qtot = plsc.load_gather(qbuf, [bvec])
                for j in range(1, LANES):
                    stot = stot + plsc.load_gather(sbuf, [bvec + j])
                    qtot = qtot + plsc.load_gather(qbuf, [bvec + j])
                mean = stot * (1.0 / DIM)
                var = qtot * (1.0 / DIM) - mean * mean
                rstd = _rsqrt(var + EPS)
                t0 = k * LANES
                for j in range(LANES):
                    mv = jnp.full((LANES,), mean[j], jnp.float32)
                    rv = jnp.full((LANES,), rstd[j], jnp.float32)
                    t = t0 + j
                    wrap = (t >= S).astype(jnp.int32)
                    row = t - wrap * S
                    col = wrap * DIM
                    for c in range(NV):
                        e = wbuf[row, pl.ds(col + c * LANES, LANES)]
                        wbuf[row, pl.ds(col + c * LANES, LANES)] = (
                            (e - mv) * rv * g_regs[c] + b_regs[c])

        pltpu.sync_copy(x_hbm.at[base], idx0)
        fire_gather(idx0)
        fire_idx(base + 1, idx1, isem1)

        def pair_body(p, carry):
            gA = 2 * p
            cA = base + gA
            cB = cA + 1
            drain_gather(idx0)

            @pl.when(p < NP - 1)
            def _():
                fire_idx(cA + 2, idx0, isem0)

            @pl.when(p > 0)
            def _():
                pltpu.make_async_copy(wbuf0, out_hbm.at[cA - 2], osem0).wait()

            pass1(wbuf0)
            drain_idx(cB, idx1, isem1)
            fire_gather(idx1)
            pass2(wbuf0)
            pltpu.async_copy(wbuf0, out_hbm.at[cA], osem0)

            drain_gather(idx1)

            @pl.when(p < NP - 1)
            def _():
                fire_idx(cB + 2, idx1, isem1)

            @pl.when(p > 0)
            def _():
                pltpu.make_async_copy(wbuf1, out_hbm.at[cB - 2], osem1).wait()

            pass1(wbuf1)

            @pl.when(p < NP - 1)
            def _():
                drain_idx(cA + 2, idx0, isem0)
                fire_gather(idx0)

            pass2(wbuf1)
            pltpu.async_copy(wbuf1, out_hbm.at[cB], osem1)
            return carry

        lax.fori_loop(0, NP, pair_body, 0)
        pltpu.make_async_copy(wbuf0, out_hbm.at[base + CPW - 2], osem0).wait()
        pltpu.make_async_copy(wbuf1, out_hbm.at[base + CPW - 1], osem1).wait()

    mid = sc_fn(x, W, P, gamma, beta)

    TCB = 8

    def tc_body(in_ref, out_ref):
        for j in range(TCB):
            out_ref[2 * j] = in_ref[j, :, :DIM]
            out_ref[2 * j + 1] = in_ref[j, :, DIM:]

    out = pl.pallas_call(
        tc_body,
        out_shape=jax.ShapeDtypeStruct((B, S, DIM), jnp.float32),
        grid=(NCHUNK // TCB,),
        in_specs=[pl.BlockSpec((TCB, S, 2 * DIM), lambda i: (i, 0, 0))],
        out_specs=pl.BlockSpec((2 * TCB, S, DIM), lambda i: (i, 0, 0)),
    )(mid)
    return out


# speedup vs baseline: 1.0852x; 1.0852x over previous
import functools

import jax
import jax.numpy as jnp
from jax import lax
from jax.experimental import pallas as pl
from jax.experimental.pallas import tpu as pltpu
from jax.experimental.pallas import tpu_sc as plsc

DIM = 64
EPS = 1e-12
B, S = 4096, 200
NC, NS = 2, 16
NW = NC * NS
CHUNK_SEQ = 2
CT = CHUNK_SEQ * S
NCHUNK = B // CHUNK_SEQ
CPW = NCHUNK // NW
NP = CPW // 2
NIDX = 4
IDXW = CT // NIDX
LANES = 16
NV = DIM // LANES
GPC = CT // LANES


def _rsqrt(v):
    vi = lax.bitcast_convert_type(v, jnp.int32)
    yi = jnp.int32(0x5F3759DF) - lax.shift_right_arithmetic(vi, 1)
    y = lax.bitcast_convert_type(yi, jnp.float32)
    h = v * 0.5
    for _ in range(3):
        y = y * (1.5 - h * y * y)
    return y


def kernel(x, W, P, gamma, beta):
    x = x.astype(jnp.int32).reshape(NCHUNK, NIDX, IDXW)
    mesh = plsc.VectorSubcoreMesh(core_axis_name="c", subcore_axis_name="s")

    @functools.partial(
        pl.kernel,
        out_type=jax.ShapeDtypeStruct((NCHUNK, S, 2 * DIM), jnp.float32),
        mesh=mesh,
        scratch_types=[
            pltpu.VMEM((NIDX, IDXW), jnp.int32),
            pltpu.VMEM((NIDX, IDXW), jnp.int32),
            pltpu.VMEM((CT, DIM), jnp.float32),
            pltpu.VMEM((S, 2 * DIM), jnp.float32),
            pltpu.VMEM((S, 2 * DIM), jnp.float32),
            pltpu.VMEM((S, DIM), jnp.float32),
            pltpu.VMEM((DIM,), jnp.float32),
            pltpu.VMEM((DIM,), jnp.float32),
            pltpu.VMEM((CT * 17,), jnp.float32),
            pltpu.VMEM((CT * 17,), jnp.float32),
            pltpu.SemaphoreType.DMA,
            pltpu.SemaphoreType.DMA,
            pltpu.SemaphoreType.DMA,
            pltpu.SemaphoreType.DMA,
            pltpu.SemaphoreType.DMA,
        ],
        compiler_params=pltpu.CompilerParams(needs_layout_passes=False,
                                             use_tc_tiling_on_sc=False),
    )
    def sc_fn(x_hbm, w_hbm, p_hbm, g_hbm, b_hbm, out_hbm,
              idx0, idx1, gbuf, wbuf0, wbuf1, p_v, g_v, b_v,
              sbuf, qbuf, gsem, osem0, osem1, isem0, isem1):
        wid = lax.axis_index("s") * NC + lax.axis_index("c")
        base = wid * CPW
        pltpu.sync_copy(p_hbm, p_v)
        pltpu.sync_copy(g_hbm, g_v)
        pltpu.sync_copy(b_hbm, b_v)
        g_regs = [g_v[pl.ds(c * LANES, LANES)] for c in range(NV)]
        b_regs = [b_v[pl.ds(c * LANES, LANES)] for c in range(NV)]

        def fire_gather(idx_b):
            for j in range(NIDX):
                pltpu.async_copy(w_hbm.at[idx_b.at[j]],
                                 gbuf.at[pl.ds(j * IDXW, IDXW)], gsem)

        def drain_gather(idx_b):
            for j in range(NIDX):
                pltpu.make_async_copy(w_hbm.at[idx_b.at[j]],
                                      gbuf.at[pl.ds(j * IDXW, IDXW)],
                                      gsem).wait()

        def fire_idx(cg, idx_b, isem):
            pltpu.async_copy(x_hbm.at[cg], idx_b, isem)

        def drain_idx(cg, idx_b, isem):
            pltpu.make_async_copy(x_hbm.at[cg], idx_b, isem).wait()

        iota17 = lax.iota(jnp.int32, LANES) * 17

        def pass1(wbuf):
            @plsc.parallel_loop(0, S, unroll=4)
            def body(si):
                p = [p_v[si, pl.ds(c * LANES, LANES)] for c in range(NV)]
                row = lax.shift_right_logical(si, 1)
                col = (si & 1) * DIM
                for r2 in range(CHUNK_SEQ):
                    t = r2 * S + si
                    e = [gbuf[t, pl.ds(c * LANES, LANES)] + p[c]
                         for c in range(NV)]
                    for c in range(NV):
                        wbuf[r2 * (S // 2) + row,
                             pl.ds(col + c * LANES, LANES)] = e[c]
                    s4 = (e[0] + e[1]) + (e[2] + e[3])
                    q4 = (e[0] * e[0] + e[1] * e[1]) + (e[2] * e[2] + e[3] * e[3])
                    sbuf[pl.ds(t * 17, LANES)] = s4
                    qbuf[pl.ds(t * 17, LANES)] = q4

        def pass2(wbuf):
            @plsc.parallel_loop(0, GPC, unroll=2)
            def body(k):
                bvec = iota17 + k * (LANES * 17)
                stot = plsc.load_gather(sbuf, [bvec])
                qtot = plsc.load_gather(qbuf, [bvec])
                for j in range(1, LANES):
                    stot = stot + plsc.load_gather(sbuf, [bvec + j])
                    qtot = qtot + plsc.load_gather(qbuf, [bvec + j])
                mean = stot * (1.0 / DIM)
                var = qtot * (1.0 / DIM) - mean * mean
                rstd = _rsqrt(var + EPS)
                row0 = k * (LANES // 2)
                for j in range(LANES):
                    mv = jnp.full((LANES,), mean[j], jnp.float32)
                    rv = jnp.full((LANES,), rstd[j], jnp.float32)
                    col = (j & 1) * DIM
                    for c in range(NV):
                        e = wbuf[row0 + j // 2, pl.ds(col + c * LANES, LANES)]
                        wbuf[row0 + j // 2, pl.ds(col + c * LANES, LANES)] = (
                            (e - mv) * rv * g_regs[c] + b_regs[c])

        pltpu.sync_copy(x_hbm.at[base], idx0)
        fire_gather(idx0)
        fire_idx(base + 1, idx1, isem1)

        def pair_body(p, carry):
            gA = 2 * p
            cA = base + gA
            cB = cA + 1
            drain_gather(idx0)

            @pl.when(p < NP - 1)
            def _():
                fire_idx(cA + 2, idx0, isem0)

            @pl.when(p > 0)
            def _():
                pltpu.make_async_copy(wbuf0, out_hbm.at[cA - 2], osem0).wait()

            pass1(wbuf0)
            drain_idx(cB, idx1, isem1)
            fire_gather(idx1)
            pass2(wbuf0)
            pltpu.async_copy(wbuf0, out_hbm.at[cA], osem0)

            drain_gather(idx1)

            @pl.when(p < NP - 1)
            def _():
                fire_idx(cB + 2, idx1, isem1)

            @pl.when(p > 0)
            def _():
                pltpu.make_async_copy(wbuf1, out_hbm.at[cB - 2], osem1).wait()

            pass1(wbuf1)

            @pl.when(p < NP - 1)
            def _():
                drain_idx(cA + 2, idx0, isem0)
                fire_gather(idx0)

            pass2(wbuf1)
            pltpu.async_copy(wbuf1, out_hbm.at[cB], osem1)
            return carry

        lax.fori_loop(0, NP, pair_body, 0)
        pltpu.make_async_copy(wbuf0, out_hbm.at[base + CPW - 2], osem0).wait()
        pltpu.make_async_copy(wbuf1, out_hbm.at[base + CPW - 1], osem1).wait()

    out = sc_fn(x, W, P, gamma, beta)
    return out.reshape(B, S, DIM)
